# baseline (speedup 1.0000x reference)
import functools

import jax
import jax.numpy as jnp
from jax import lax
from jax.experimental import pallas as pl
from jax.experimental.pallas import tpu as pltpu
from jax.experimental.pallas import tpu_sc as plsc

B, H, L, D, C = 2, 16, 4096, 64, 512
COMMITMENT = 0.0001
HB = 2

_info = plsc.get_sparse_core_info()
_NC, _NS, _LANES = _info.num_cores, _info.num_subcores, _info.num_lanes
_NW = _NC * _NS
_N = B * H * L
_CHUNK = _N // _NW


def _tc_kernel(x_ref, m_ref, dists_ref, gidx_ref, m2_ref, part_ref):
    i = pl.program_id(0)
    acc = jnp.zeros((1, 1), jnp.float32)
    for t in range(HB):
        xb = x_ref[0, t]
        m = m_ref[t]
        n2 = jnp.sum(xb * xb, axis=-1, keepdims=True)
        xn = xb / jnp.maximum(jnp.sqrt(n2), 1e-12)
        d = jax.lax.dot_general(
            xn, m, (((1,), (1,)), ((), ())),
            preferred_element_type=jnp.float32)
        dists_ref[0, t] = d
        maxv = jnp.max(d, axis=-1, keepdims=True)
        iota = lax.broadcasted_iota(jnp.int32, (L, C), 1)
        idx = jnp.min(jnp.where(d == maxv, iota, C), axis=-1)
        h = (i % (H // HB)) * HB + t
        gidx_ref[0, t, 0] = idx + h * C
        m2_ref[t, 0] = jnp.sum(m * m, axis=-1)
        xn2 = jnp.sum(xn * xn, axis=-1)
        acc = acc + (jnp.sum(xn2) - 2.0 * jnp.sum(maxv)).reshape(1, 1)

    @pl.when(i == 0)
    def _init():
        part_ref[0] = jnp.zeros((1, 1), jnp.float32)

    part_ref[0] += acc


def _sc_loss_kernel(m2_hbm, gidx_hbm, out_hbm, m2_v, idx_v, acc_v):
    wid = lax.axis_index("s") * _NC + lax.axis_index("c")
    base = wid * _CHUNK
    pltpu.sync_copy(m2_hbm, m2_v)
    pltpu.sync_copy(gidx_hbm.at[pl.ds(base, _CHUNK)], idx_v)

    def body(j, acc):
        iv = idx_v[pl.ds(j * _LANES, _LANES)]
        return acc + plsc.load_gather(m2_v, [iv])

    acc = lax.fori_loop(0, _CHUNK // _LANES, body,
                        jnp.zeros((_LANES,), jnp.float32))
    acc_v[...] = acc
    pltpu.sync_copy(acc_v, out_hbm.at[wid])


@jax.jit
def kernel(x, means):
    G = B * H // HB
    HG = H // HB
    dists, gidx, m2, tc_part = pl.pallas_call(
        _tc_kernel,
        grid=(G,),
        in_specs=[
            pl.BlockSpec((1, HB, L, D), lambda i: (i // HG, i % HG, 0, 0)),
            pl.BlockSpec((HB, C, D), lambda i: (i % HG, 0, 0)),
        ],
        out_specs=[
            pl.BlockSpec((1, HB, L, C), lambda i: (i // HG, i % HG, 0, 0)),
            pl.BlockSpec((1, HB, 1, L), lambda i: (i // HG, i % HG, 0, 0)),
            pl.BlockSpec((HB, 1, C), lambda i: (i % HG, 0, 0)),
            pl.BlockSpec((1, 1, 1), lambda i: (0, 0, 0)),
        ],
        out_shape=[
            jax.ShapeDtypeStruct((B, H, L, C), jnp.float32),
            jax.ShapeDtypeStruct((B, H, 1, L), jnp.int32),
            jax.ShapeDtypeStruct((H, 1, C), jnp.float32),
            jax.ShapeDtypeStruct((1, 1, 1), jnp.float32),
        ],
        compiler_params=pltpu.CompilerParams(
            dimension_semantics=("arbitrary",)),
    )(x, means)

    mesh = plsc.VectorSubcoreMesh(core_axis_name="c", subcore_axis_name="s")
    sc_parts = functools.partial(
        pl.kernel,
        mesh=mesh,
        out_type=jax.ShapeDtypeStruct((_NW, _LANES), jnp.float32),
        scratch_types=[
            pltpu.VMEM((H * C,), jnp.float32),
            pltpu.VMEM((_CHUNK,), jnp.int32),
            pltpu.VMEM((_LANES,), jnp.float32),
        ],
        compiler_params=pltpu.CompilerParams(needs_layout_passes=False),
    )(_sc_loss_kernel)(m2.reshape(H * C), gidx.reshape(_N))

    loss = (tc_part.reshape(()) + jnp.sum(sc_parts)) * (COMMITMENT / (_N * D))
    return dists, loss


# speedup vs baseline: 1.0951x; 1.0951x over previous
import functools

import jax
import jax.numpy as jnp
from jax import lax
from jax.experimental import pallas as pl
from jax.experimental.pallas import tpu as pltpu
from jax.experimental.pallas import tpu_sc as plsc

B, H, L, D, C = 2, 16, 4096, 64, 512
COMMITMENT = 0.0001
HB = 2

_info = plsc.get_sparse_core_info()
_NC, _NS, _LANES = _info.num_cores, _info.num_subcores, _info.num_lanes
_NW = _NC * _NS
_N = B * H * L
_CHUNK = _N // _NW


def _tc_kernel(x_ref, m_ref, dists_ref, gidx_ref, m2_ref, part_ref):
    i = pl.program_id(0)
    acc = jnp.zeros((1, 1), jnp.float32)
    for t in range(HB):
        xb = x_ref[0, t]
        m = m_ref[t]
        n2 = jnp.sum(xb * xb, axis=-1, keepdims=True)
        xn = xb / jnp.maximum(jnp.sqrt(n2), 1e-12)
        d = jax.lax.dot_general(
            xn, m, (((1,), (1,)), ((), ())),
            preferred_element_type=jnp.float32)
        dists_ref[0, t] = d
        maxv = jnp.max(d, axis=-1, keepdims=True)
        iota = lax.broadcasted_iota(jnp.int32, (L, C), 1)
        idx = jnp.min(jnp.where(d == maxv, iota, C),
                      axis=-1, keepdims=True)
        h = (i % (H // HB)) * HB + t
        gidx_ref[0, t] = idx + h * C
        m2_ref[t, 0] = jnp.sum(m * m, axis=-1)
        xn2 = jnp.sum(xn * xn, axis=-1)
        acc = acc + (jnp.sum(xn2) - 2.0 * jnp.sum(maxv)).reshape(1, 1)

    @pl.when(i == 0)
    def _init():
        part_ref[0] = jnp.zeros((1, 1), jnp.float32)

    part_ref[0] += acc


def _sc_loss_kernel(m2_hbm, gidx_hbm, out_hbm, m2_v, idx_v, acc_v):
    wid = lax.axis_index("s") * _NC + lax.axis_index("c")
    base = wid * _CHUNK
    pltpu.sync_copy(m2_hbm, m2_v)
    pltpu.sync_copy(gidx_hbm.at[pl.ds(base, _CHUNK)], idx_v)

    def body(j, acc):
        iv = idx_v[pl.ds(j * _LANES, _LANES)]
        return acc + plsc.load_gather(m2_v, [iv])

    acc = lax.fori_loop(0, _CHUNK // _LANES, body,
                        jnp.zeros((_LANES,), jnp.float32))
    acc_v[...] = acc
    pltpu.sync_copy(acc_v, out_hbm.at[wid])


@jax.jit
def kernel(x, means):
    G = B * H // HB
    HG = H // HB
    dists, gidx, m2, tc_part = pl.pallas_call(
        _tc_kernel,
        grid=(G,),
        in_specs=[
            pl.BlockSpec((1, HB, L, D), lambda i: (i // HG, i % HG, 0, 0)),
            pl.BlockSpec((HB, C, D), lambda i: (i % HG, 0, 0)),
        ],
        out_specs=[
            pl.BlockSpec((1, HB, L, C), lambda i: (i // HG, i % HG, 0, 0)),
            pl.BlockSpec((1, HB, L, 1), lambda i: (i // HG, i % HG, 0, 0)),
            pl.BlockSpec((HB, 1, C), lambda i: (i % HG, 0, 0)),
            pl.BlockSpec((1, 1, 1), lambda i: (0, 0, 0)),
        ],
        out_shape=[
            jax.ShapeDtypeStruct((B, H, L, C), jnp.float32),
            jax.ShapeDtypeStruct((B, H, L, 1), jnp.int32),
            jax.ShapeDtypeStruct((H, 1, C), jnp.float32),
            jax.ShapeDtypeStruct((1, 1, 1), jnp.float32),
        ],
        compiler_params=pltpu.CompilerParams(
            dimension_semantics=("arbitrary",)),
    )(x, means)

    mesh = plsc.VectorSubcoreMesh(core_axis_name="c", subcore_axis_name="s")
    sc_parts = functools.partial(
        pl.kernel,
        mesh=mesh,
        out_type=jax.ShapeDtypeStruct((_NW, _LANES), jnp.float32),
        scratch_types=[
            pltpu.VMEM((H * C,), jnp.float32),
            pltpu.VMEM((_CHUNK,), jnp.int32),
            pltpu.VMEM((_LANES,), jnp.float32),
        ],
        compiler_params=pltpu.CompilerParams(needs_layout_passes=False),
    )(_sc_loss_kernel)(m2.reshape(H * C), gidx.reshape(_N))

    loss = (tc_part.reshape(()) + jnp.sum(sc_parts)) * (COMMITMENT / (_N * D))
    return dists, loss


# speedup vs baseline: 1.2254x; 1.1190x over previous
import functools

import jax
import jax.numpy as jnp
from jax import lax
from jax.experimental import pallas as pl
from jax.experimental.pallas import tpu as pltpu
from jax.experimental.pallas import tpu_sc as plsc

B, H, L, D, C = 2, 16, 4096, 64, 512
COMMITMENT = 0.0001
HB = 2

_info = plsc.get_sparse_core_info()
_NC, _NS, _LANES = _info.num_cores, _info.num_subcores, _info.num_lanes
_NW = _NC * _NS
_N = B * H * L
_CHUNK = _N // _NW


def _tc_kernel(x_ref, m_ref, dists_ref, gidx_ref, m2_ref, part_ref):
    i = pl.program_id(0)
    acc = jnp.zeros((1, 1), jnp.float32)
    for t in range(HB):
        xb = x_ref[0, t]
        m = m_ref[t]
        n2 = jnp.sum(xb * xb, axis=-1, keepdims=True)
        xn = xb / jnp.maximum(jnp.sqrt(n2), 1e-12)
        d = jax.lax.dot_general(
            xn, m, (((1,), (1,)), ((), ())),
            preferred_element_type=jnp.float32)
        dists_ref[0, t] = d
        dT = jax.lax.dot_general(
            m, xn, (((1,), (1,)), ((), ())),
            preferred_element_type=jnp.float32)
        maxvT = jnp.max(dT, axis=0, keepdims=True)
        iotaT = lax.broadcasted_iota(jnp.int32, (C, L), 0)
        idxT = jnp.min(jnp.where(dT == maxvT, iotaT, C), axis=0)
        h = (i % (H // HB)) * HB + t
        gidx_ref[0, t] = idxT + h * C
        m2_ref[t, 0] = jnp.sum(m * m, axis=-1)
        xn2 = jnp.sum(xn * xn, axis=-1)
        acc = acc + (jnp.sum(xn2) - 2.0 * jnp.sum(maxvT)).reshape(1, 1)

    @pl.when(i == 0)
    def _init():
        part_ref[0] = jnp.zeros((1, 1), jnp.float32)

    part_ref[0] += acc


def _sc_loss_kernel(m2_hbm, gidx_hbm, out_hbm, m2_v, idx_v, acc_v):
    wid = lax.axis_index("s") * _NC + lax.axis_index("c")
    base = wid * _CHUNK
    pltpu.sync_copy(m2_hbm, m2_v)
    pltpu.sync_copy(gidx_hbm.at[pl.ds(base, _CHUNK)], idx_v)

    def body(j, acc):
        iv = idx_v[pl.ds(j * _LANES, _LANES)]
        return acc + plsc.load_gather(m2_v, [iv])

    acc = lax.fori_loop(0, _CHUNK // _LANES, body,
                        jnp.zeros((_LANES,), jnp.float32))
    acc_v[...] = acc
    pltpu.sync_copy(acc_v, out_hbm.at[wid])


@jax.jit
def kernel(x, means):
    G = B * H // HB
    HG = H // HB
    dists, gidx, m2, tc_part = pl.pallas_call(
        _tc_kernel,
        grid=(G,),
        in_specs=[
            pl.BlockSpec((1, HB, L, D), lambda i: (i // HG, i % HG, 0, 0)),
            pl.BlockSpec((HB, C, D), lambda i: (i % HG, 0, 0)),
        ],
        out_specs=[
            pl.BlockSpec((1, HB, L, C), lambda i: (i // HG, i % HG, 0, 0)),
            pl.BlockSpec((1, HB, L), lambda i: (i, 0, 0)),
            pl.BlockSpec((HB, 1, C), lambda i: (i % HG, 0, 0)),
            pl.BlockSpec((1, 1, 1), lambda i: (0, 0, 0)),
        ],
        out_shape=[
            jax.ShapeDtypeStruct((B, H, L, C), jnp.float32),
            jax.ShapeDtypeStruct((B * H // HB, HB, L), jnp.int32),
            jax.ShapeDtypeStruct((H, 1, C), jnp.float32),
            jax.ShapeDtypeStruct((1, 1, 1), jnp.float32),
        ],
        compiler_params=pltpu.CompilerParams(
            dimension_semantics=("arbitrary",)),
    )(x, means)

    mesh = plsc.VectorSubcoreMesh(core_axis_name="c", subcore_axis_name="s")
    sc_parts = functools.partial(
        pl.kernel,
        mesh=mesh,
        out_type=jax.ShapeDtypeStruct((_NW, _LANES), jnp.float32),
        scratch_types=[
            pltpu.VMEM((H * C,), jnp.float32),
            pltpu.VMEM((_CHUNK,), jnp.int32),
            pltpu.VMEM((_LANES,), jnp.float32),
        ],
        compiler_params=pltpu.CompilerParams(needs_layout_passes=False),
    )(_sc_loss_kernel)(m2.reshape(H * C), gidx.reshape(_N))

    loss = (tc_part.reshape(()) + jnp.sum(sc_parts)) * (COMMITMENT / (_N * D))
    return dists, loss


# speedup vs baseline: 1.8136x; 1.4801x over previous
import functools

import jax
import jax.numpy as jnp
from jax import lax
from jax.experimental import pallas as pl
from jax.experimental.pallas import tpu as pltpu
from jax.experimental.pallas import tpu_sc as plsc

B, H, L, D, C = 2, 16, 4096, 64, 512
COMMITMENT = 0.0001
HB = 2

_info = plsc.get_sparse_core_info()
_NC, _NS, _LANES = _info.num_cores, _info.num_subcores, _info.num_lanes
_NW = _NC * _NS
_N = B * H * L
_CHUNK = _N // _NW


def _tc_kernel(xT_ref, mT_ref, dists_ref, gidx_ref, m2_ref, part_ref):
    i = pl.program_id(0)
    acc = jnp.zeros((1, 1), jnp.float32)
    for t in range(HB):
        xbT = xT_ref[0, t]
        mT = mT_ref[t]
        n2T = jnp.sum(xbT * xbT, axis=0, keepdims=True)
        invT = 1.0 / jnp.maximum(jnp.sqrt(n2T), 1e-12)
        xnT = xbT * invT
        d = jax.lax.dot_general(
            xnT, mT, (((0,), (0,)), ((), ())),
            preferred_element_type=jnp.float32)
        dists_ref[0, t] = d
        dT = jax.lax.dot_general(
            mT, xnT, (((0,), (0,)), ((), ())),
            preferred_element_type=jnp.float32)
        maxvT = jnp.max(dT, axis=0, keepdims=True)
        iotaT = lax.broadcasted_iota(jnp.int32, (C, L), 0)
        idxT = jnp.min(jnp.where(dT == maxvT, iotaT, C), axis=0)
        h = (i % (H // HB)) * HB + t
        gidx_ref[0, t] = idxT + h * C
        m2_ref[t, 0] = jnp.sum(mT * mT, axis=0)
        acc = acc + (jnp.sum(n2T * invT * invT)
                     - 2.0 * jnp.sum(maxvT)).reshape(1, 1)

    @pl.when(i == 0)
    def _init():
        part_ref[0] = jnp.zeros((1, 1), jnp.float32)

    part_ref[0] += acc


def _sc_loss_kernel(m2_hbm, gidx_hbm, out_hbm, m2_v, idx_v, acc_v):
    wid = lax.axis_index("s") * _NC + lax.axis_index("c")
    base = wid * _CHUNK
    pltpu.sync_copy(m2_hbm, m2_v)
    pltpu.sync_copy(gidx_hbm.at[pl.ds(base, _CHUNK)], idx_v)

    def body(j, acc):
        iv = idx_v[pl.ds(j * _LANES, _LANES)]
        return acc + plsc.load_gather(m2_v, [iv])

    acc = lax.fori_loop(0, _CHUNK // _LANES, body,
                        jnp.zeros((_LANES,), jnp.float32))
    acc_v[...] = acc
    pltpu.sync_copy(acc_v, out_hbm.at[wid])


@jax.jit
def kernel(x, means):
    G = B * H // HB
    HG = H // HB
    dists, gidx, m2, tc_part = pl.pallas_call(
        _tc_kernel,
        grid=(G,),
        in_specs=[
            pl.BlockSpec((1, HB, D, L), lambda i: (i // HG, i % HG, 0, 0)),
            pl.BlockSpec((HB, D, C), lambda i: (i % HG, 0, 0)),
        ],
        out_specs=[
            pl.BlockSpec((1, HB, L, C), lambda i: (i // HG, i % HG, 0, 0)),
            pl.BlockSpec((1, HB, L), lambda i: (i, 0, 0)),
            pl.BlockSpec((HB, 1, C), lambda i: (i % HG, 0, 0)),
            pl.BlockSpec((1, 1, 1), lambda i: (0, 0, 0)),
        ],
        out_shape=[
            jax.ShapeDtypeStruct((B, H, L, C), jnp.float32),
            jax.ShapeDtypeStruct((B * H // HB, HB, L), jnp.int32),
            jax.ShapeDtypeStruct((H, 1, C), jnp.float32),
            jax.ShapeDtypeStruct((1, 1, 1), jnp.float32),
        ],
        compiler_params=pltpu.CompilerParams(
            dimension_semantics=("arbitrary",)),
    )(jnp.swapaxes(x, 2, 3), jnp.swapaxes(means, 1, 2))

    mesh = plsc.VectorSubcoreMesh(core_axis_name="c", subcore_axis_name="s")
    sc_parts = functools.partial(
        pl.kernel,
        mesh=mesh,
        out_type=jax.ShapeDtypeStruct((_NW, _LANES), jnp.float32),
        scratch_types=[
            pltpu.VMEM((H * C,), jnp.float32),
            pltpu.VMEM((_CHUNK,), jnp.int32),
            pltpu.VMEM((_LANES,), jnp.float32),
        ],
        compiler_params=pltpu.CompilerParams(needs_layout_passes=False),
    )(_sc_loss_kernel)(m2.reshape(H * C), gidx.reshape(_N))

    loss = (tc_part.reshape(()) + jnp.sum(sc_parts)) * (COMMITMENT / (_N * D))
    return dists, loss
